# baseline (device time: 21974 ns/iter reference)
import jax
import jax.numpy as jnp
from jax import lax
from jax.experimental import pallas as pl
from jax.experimental.pallas import tpu as pltpu

CH = 32
Y_CHUNKS = 17
FWD_CHUNKS = 15


def kernel(x):
    m, n = x.shape
    half = n // 2
    assert (Y_CHUNKS + FWD_CHUNKS) * CH == m

    def body(x_ref, out_ref, y_send, y_recv, x_send, x_recv):
        my_x = lax.axis_index("x")
        my_y = lax.axis_index("y")
        peer_y = 1 - my_y
        peer_x = 1 - my_x

        base = my_x * (m - CH)
        sign = 1 - 2 * my_x

        barrier = pltpu.get_barrier_semaphore()
        for dev in ((my_x, peer_y), (peer_x, my_y)):
            pl.semaphore_signal(
                barrier, inc=1, device_id=dev,
                device_id_type=pl.DeviceIdType.MESH,
            )
        pl.semaphore_wait(barrier, 2)

        y_rdmas = []
        for c in range(Y_CHUNKS):
            r = base + sign * (c * CH)
            rdma = pltpu.make_async_remote_copy(
                src_ref=x_ref.at[pl.ds(r, CH), pl.ds(peer_y * half, half)],
                dst_ref=out_ref.at[pl.ds(my_y * m + r, CH), :],
                send_sem=y_send.at[c],
                recv_sem=y_recv.at[c],
                device_id=(my_x, peer_y),
                device_id_type=pl.DeviceIdType.MESH,
            )
            rdma.start()
            y_rdmas.append(rdma)

        x_rdmas = []
        sem_i = 0
        for c in range(FWD_CHUNKS):
            y_rdmas[c].wait_recv()
            r = peer_y * m + base + sign * (c * CH)
            if c < FWD_CHUNKS - 1:
                subs = [(0, CH)]
            else:
                subs = [(0, CH // 2), (CH // 2, CH // 2)]
            for off, sz in subs:
                fwd = pltpu.make_async_remote_copy(
                    src_ref=out_ref.at[pl.ds(r + off, sz), :],
                    dst_ref=out_ref.at[pl.ds(r + off, sz), :],
                    send_sem=x_send.at[sem_i],
                    recv_sem=x_recv.at[sem_i],
                    device_id=(peer_x, my_y),
                    device_id_type=pl.DeviceIdType.MESH,
                )
                fwd.start()
                x_rdmas.append(fwd)
                sem_i += 1

        out_ref[pl.ds(my_y * m, m), :] = x_ref[:, pl.ds(my_y * half, half)]

        for c in range(FWD_CHUNKS, Y_CHUNKS):
            y_rdmas[c].wait_recv()
        for c in range(Y_CHUNKS):
            y_rdmas[c].wait_send()
        for fwd in x_rdmas:
            fwd.wait_recv()
            fwd.wait_send()

    out_shape = jax.ShapeDtypeStruct((2 * m, half), x.dtype)
    return pl.pallas_call(
        body,
        out_shape=out_shape,
        in_specs=[pl.BlockSpec(memory_space=pltpu.VMEM)],
        out_specs=pl.BlockSpec(memory_space=pltpu.VMEM),
        scratch_shapes=[
            pltpu.SemaphoreType.DMA((Y_CHUNKS,)),
            pltpu.SemaphoreType.DMA((Y_CHUNKS,)),
            pltpu.SemaphoreType.DMA((FWD_CHUNKS + 1,)),
            pltpu.SemaphoreType.DMA((FWD_CHUNKS + 1,)),
        ],
        compiler_params=pltpu.CompilerParams(collective_id=0),
    )(x)


# device time: 21932 ns/iter; 1.0019x vs baseline; 1.0019x over previous
import jax
import jax.numpy as jnp
from jax import lax
from jax.experimental import pallas as pl
from jax.experimental.pallas import tpu as pltpu

CH = 32
Y_CHUNKS = 17
FWD_CHUNKS = 15


def kernel(x):
    m, n = x.shape
    half = n // 2
    assert (Y_CHUNKS + FWD_CHUNKS) * CH == m

    def body(x_ref, out_ref, y_send, y_recv, x_send, x_recv):
        my_x = lax.axis_index("x")
        my_y = lax.axis_index("y")
        peer_y = 1 - my_y
        peer_x = 1 - my_x

        base = my_x * (m - CH)
        sign = 1 - 2 * my_x

        barrier = pltpu.get_barrier_semaphore()
        for dev in ((my_x, peer_y), (peer_x, my_y)):
            pl.semaphore_signal(
                barrier, inc=1, device_id=dev,
                device_id_type=pl.DeviceIdType.MESH,
            )
        pl.semaphore_wait(barrier, 2)

        y_rdmas = []
        for c in range(Y_CHUNKS):
            r = base + sign * (c * CH)
            rdma = pltpu.make_async_remote_copy(
                src_ref=x_ref.at[pl.ds(r, CH), pl.ds(peer_y * half, half)],
                dst_ref=out_ref.at[pl.ds(my_y * m + r, CH), :],
                send_sem=y_send.at[c],
                recv_sem=y_recv.at[c],
                device_id=(my_x, peer_y),
                device_id_type=pl.DeviceIdType.MESH,
            )
            rdma.start()
            y_rdmas.append(rdma)

        x_rdmas = []
        for c in range(FWD_CHUNKS):
            y_rdmas[c].wait_recv()
            r = peer_y * m + base + sign * (c * CH)
            fwd = pltpu.make_async_remote_copy(
                src_ref=out_ref.at[pl.ds(r, CH), :],
                dst_ref=out_ref.at[pl.ds(r, CH), :],
                send_sem=x_send.at[c],
                recv_sem=x_recv.at[c],
                device_id=(peer_x, my_y),
                device_id_type=pl.DeviceIdType.MESH,
            )
            fwd.start()
            x_rdmas.append(fwd)

        out_ref[pl.ds(my_y * m, m), :] = x_ref[:, pl.ds(my_y * half, half)]

        for c in range(FWD_CHUNKS, Y_CHUNKS):
            y_rdmas[c].wait_recv()
        for c in range(Y_CHUNKS):
            y_rdmas[c].wait_send()
        for c in range(FWD_CHUNKS):
            x_rdmas[c].wait_recv()
            x_rdmas[c].wait_send()

    out_shape = jax.ShapeDtypeStruct((2 * m, half), x.dtype)
    return pl.pallas_call(
        body,
        out_shape=out_shape,
        in_specs=[pl.BlockSpec(memory_space=pltpu.VMEM)],
        out_specs=pl.BlockSpec(memory_space=pltpu.VMEM),
        scratch_shapes=[
            pltpu.SemaphoreType.DMA((Y_CHUNKS,)),
            pltpu.SemaphoreType.DMA((Y_CHUNKS,)),
            pltpu.SemaphoreType.DMA((FWD_CHUNKS,)),
            pltpu.SemaphoreType.DMA((FWD_CHUNKS,)),
        ],
        compiler_params=pltpu.CompilerParams(collective_id=0),
    )(x)
